# Initial kernel scaffold; baseline (speedup 1.0000x reference)
import jax
import jax.numpy as jnp
from jax.experimental import pallas as pl


def kernel(scores_pred, boxes_pred):
    raise NotImplementedError("write your pallas kernel here")


# speedup vs baseline: 2.9454x; 2.9454x over previous
import jax
import jax.numpy as jnp
from jax import lax
from jax.experimental import pallas as pl
from jax.experimental.pallas import tpu as pltpu

_SCORE_THRESHOLD = 0.3
_IOU_THRESHOLD = 0.5
_TOP_K = 200
_MAX_NMS = 100
_B, _N, _C = 8, 5000, 21
_CP = 24
_NP = 5120
_NEG = -1e30
_BIGI = 1 << 30


def _nms_body(sc_ref, bx_ref, out_ref, sm, selS, selY0, selX0, selY1, selX1,
              merged):
    y0r = bx_ref[0, 0:1, :]
    x0r = bx_ref[0, 1:2, :]
    y1r = bx_ref[0, 2:3, :]
    x1r = bx_ref[0, 3:4, :]
    area2 = jnp.maximum(y1r - y0r, 0.0) * jnp.maximum(x1r - x0r, 0.0)

    sc = sc_ref[0]
    sm[:] = jnp.where(sc > _SCORE_THRESHOLD, sc, _NEG)
    selS[:] = jnp.full((_CP, 128), _NEG, jnp.float32)
    selY0[:] = jnp.zeros((_CP, 128), jnp.float32)
    selX0[:] = jnp.zeros((_CP, 128), jnp.float32)
    selY1[:] = jnp.zeros((_CP, 128), jnp.float32)
    selX1[:] = jnp.zeros((_CP, 128), jnp.float32)

    lane_iota = lax.broadcasted_iota(jnp.int32, (_CP, _NP), 1)
    lane_sel = lax.broadcasted_iota(jnp.int32, (_CP, 128), 1)

    def step(t, _):
        s = sm[:]
        best = jnp.max(s, axis=1, keepdims=True)
        eq = s == best
        idx = jnp.min(jnp.where(eq, lane_iota, _BIGI), axis=1, keepdims=True)
        oh = lane_iota == idx
        ohf = oh.astype(jnp.float32)
        by0 = jnp.sum(ohf * y0r, axis=1, keepdims=True)
        bx0 = jnp.sum(ohf * x0r, axis=1, keepdims=True)
        by1 = jnp.sum(ohf * y1r, axis=1, keepdims=True)
        bx1 = jnp.sum(ohf * x1r, axis=1, keepdims=True)
        iymin = jnp.maximum(by0, y0r)
        ixmin = jnp.maximum(bx0, x0r)
        iymax = jnp.minimum(by1, y1r)
        ixmax = jnp.minimum(bx1, x1r)
        inter = jnp.maximum(iymax - iymin, 0.0) * jnp.maximum(ixmax - ixmin, 0.0)
        a1 = jnp.maximum(by1 - by0, 0.0) * jnp.maximum(bx1 - bx0, 0.0)
        union = a1 + area2 - inter
        safe_union = jnp.where(union > 0, union, 1.0)
        iou = jnp.where(union > 0, inter / safe_union, 0.0)
        supp = (iou > _IOU_THRESHOLD) | oh
        sm[:] = jnp.where(supp, _NEG, s)
        valid = best > jnp.float32(-1e29)
        vf = valid.astype(jnp.float32)
        tm = lane_sel == t
        selS[:] = jnp.where(tm, jnp.where(valid, best, 0.0), selS[:])
        selY0[:] = jnp.where(tm, vf * by0, selY0[:])
        selX0[:] = jnp.where(tm, vf * bx0, selX0[:])
        selY1[:] = jnp.where(tm, vf * by1, selY1[:])
        selX1[:] = jnp.where(tm, vf * bx1, selX1[:])
        return 0

    lax.fori_loop(0, _MAX_NMS, step, 0)

    lane128 = lax.broadcasted_iota(jnp.int32, (_CP, 128), 1)
    row_iota = lax.broadcasted_iota(jnp.int32, (_CP, 1), 0)
    sS = selS[:]
    cls_e = jnp.where(sS > 0.25, (row_iota + 1).astype(jnp.float32), 0.0)
    sY0 = selY0[:]
    sX0 = selX0[:]
    sY1 = selY1[:]
    sX1 = selX1[:]

    row8 = lax.broadcasted_iota(jnp.int32, (8, 256), 0)
    lane256 = lax.broadcasted_iota(jnp.int32, (8, 256), 1)
    merged[:] = jnp.zeros((8, 256), jnp.float32)

    def mstep(r, heads):
        hoh = lane128 == heads
        hs = jnp.sum(jnp.where(hoh, sS, 0.0), axis=1, keepdims=True)
        best = jnp.max(hs, axis=0, keepdims=True)
        flat = row_iota * _MAX_NMS + heads
        wflat = jnp.min(jnp.where(hs == best, flat, _BIGI), axis=0,
                        keepdims=True)
        wrow = flat == wflat
        woh = (wrow & hoh).astype(jnp.float32)
        vals = [jnp.sum(woh * a) for a in (cls_e, sS, sY0, sX0, sY1, sX1)]
        col = jnp.zeros((8, 256), jnp.float32)
        for k, v in enumerate(vals):
            col = col + jnp.where(row8 == k, v, 0.0)
        merged[:] = jnp.where(lane256 == r, col, merged[:])
        return heads + wrow.astype(jnp.int32)

    lax.fori_loop(0, _TOP_K, mstep, jnp.zeros((_CP, 1), jnp.int32))
    out_ref[0] = merged[:]


def _decode(scores_t, boxes_t):
    return pl.pallas_call(
        _nms_body,
        grid=(_B,),
        in_specs=[
            pl.BlockSpec((1, _CP, _NP), lambda b: (b, 0, 0)),
            pl.BlockSpec((1, 8, _NP), lambda b: (b, 0, 0)),
        ],
        out_specs=pl.BlockSpec((1, 8, 256), lambda b: (b, 0, 0)),
        out_shape=jax.ShapeDtypeStruct((_B, 8, 256), jnp.float32),
        scratch_shapes=[
            pltpu.VMEM((_CP, _NP), jnp.float32),
            pltpu.VMEM((_CP, 128), jnp.float32),
            pltpu.VMEM((_CP, 128), jnp.float32),
            pltpu.VMEM((_CP, 128), jnp.float32),
            pltpu.VMEM((_CP, 128), jnp.float32),
            pltpu.VMEM((_CP, 128), jnp.float32),
            pltpu.VMEM((8, 256), jnp.float32),
        ],
    )(scores_t, boxes_t)


def kernel(scores_pred, boxes_pred):
    scores_t = jnp.transpose(scores_pred[:, :, 1:], (0, 2, 1))
    scores_t = jnp.pad(scores_t, ((0, 0), (0, _CP - (_C - 1)),
                                  (0, _NP - _N)))
    boxes_t = jnp.transpose(boxes_pred, (0, 2, 1))
    boxes_t = jnp.pad(boxes_t, ((0, 0), (0, 4), (0, _NP - _N)))
    res = _decode(scores_t, boxes_t)
    cls = res[:, 0, :_TOP_K]
    score = res[:, 1, :_TOP_K]
    top_scores = jnp.stack([cls, score], axis=-1)
    top_boxes = jnp.transpose(res[:, 2:6, :_TOP_K], (0, 2, 1))
    return top_scores, top_boxes


# speedup vs baseline: 6.9365x; 2.3550x over previous
import jax
import jax.numpy as jnp
from jax import lax
from jax.experimental import pallas as pl
from jax.experimental.pallas import tpu as pltpu
from jax.experimental.pallas import tpu_sc as plsc

_SCORE_THRESHOLD = 0.3
_IOU_THRESHOLD = 0.5
_TOP_K = 200
_MAX_NMS = 100
_B, _N, _C = 8, 5000, 21
_CP = 24
_NP = 5120
_NB = _NP // 16
_NEG = -1e30
_BIGI = 1 << 30


def _sc_nms_body(scores_hbm, boxes_hbm, outS, outY0, outX0, outY1, outX1,
                 sw, bx, m1, m2, oS, oY0, oX0, oY1, oX1):
    cid = lax.axis_index("c")
    sid = lax.axis_index("s")
    wid = cid * 16 + sid
    b = wid % 8
    g = wid // 8

    for r in range(5):
        pltpu.sync_copy(scores_hbm.at[b, g, r], sw.at[pl.ds(r * _NP, _NP)])
    for r in range(4):
        pltpu.sync_copy(boxes_hbm.at[b, r], bx.at[pl.ds(r * _NP, _NP)])

    iota = lax.iota(jnp.int32, 16)
    zeros16 = jnp.zeros((16,), jnp.float32)
    negs16 = jnp.full((16,), _NEG, jnp.float32)
    lane0 = iota == 0

    def splat(v):
        return jnp.full((16,), v, jnp.int32)

    def class_body(ci, _carry):
        cb = ci * _NP
        co = ci * 128
        for j in range(8):
            li = co + j * 16 + iota
            for ref in (oS, oY0, oX0, oY1, oX1):
                plsc.store_scatter(ref, [li], zeros16)

        def m1_body(jv, _):
            acc = negs16
            for kk in range(16):
                idx = cb + jv * 256 + iota * 16 + kk
                v = plsc.load_gather(sw, [idx])
                v = jnp.where(v > _SCORE_THRESHOLD, v, _NEG)
                plsc.store_scatter(sw, [idx], v)
                acc = jnp.maximum(acc, v)
            plsc.store_scatter(m1, [jv * 16 + iota], acc)
            return 0

        lax.fori_loop(0, _NB // 16, m1_body, 0)

        for jv2 in range(2):
            acc = negs16
            for kk in range(16):
                idxm = jnp.minimum((jv2 * 16 + iota) * 16 + kk, _NB - 1)
                acc = jnp.maximum(acc, plsc.load_gather(m1, [idxm]))
            if jv2 == 1:
                acc = jnp.where(jv2 * 16 + iota < 20, acc, _NEG)
            plsc.store_scatter(m2, [jv2 * 16 + iota], acc)

        def m2max():
            return jnp.max(jnp.maximum(m2[0:16], m2[16:32]))

        def cond(st):
            k, gm = st
            return (k < _MAX_NMS) & (gm > jnp.float32(-1e29))

        def body(st):
            k, gm = st
            v0 = m2[0:16]
            v1 = m2[16:32]
            c0 = jnp.where(v0 == gm, iota, _BIGI)
            c1 = jnp.where(v1 == gm, iota + 16, _BIGI)
            jstar = jnp.min(jnp.minimum(c0, c1))
            mi = jstar * 16 + iota
            mv = plsc.load_gather(m1, [mi])
            bstar = jnp.min(jnp.where(mv == gm, mi, _BIGI))
            si = bstar * 16 + iota
            sv = plsc.load_gather(sw, [cb + si])
            istar = jnp.min(jnp.where(sv == gm, si, _BIGI))
            ivec = splat(istar)
            by0 = plsc.load_gather(bx, [ivec])
            bx0 = plsc.load_gather(bx, [ivec + _NP])
            by1 = plsc.load_gather(bx, [ivec + 2 * _NP])
            bx1 = plsc.load_gather(bx, [ivec + 3 * _NP])
            a1 = jnp.maximum(by1 - by0, 0.0) * jnp.maximum(bx1 - bx0, 0.0)

            nk = (k + 15) // 16

            def iou_body(j, accmax):
                ki = co + j * 16 + iota
                ky0 = plsc.load_gather(oY0, [ki])
                kx0 = plsc.load_gather(oX0, [ki])
                ky1 = plsc.load_gather(oY1, [ki])
                kx1 = plsc.load_gather(oX1, [ki])
                iymin = jnp.maximum(by0, ky0)
                ixmin = jnp.maximum(bx0, kx0)
                iymax = jnp.minimum(by1, ky1)
                ixmax = jnp.minimum(bx1, kx1)
                inter = (jnp.maximum(iymax - iymin, 0.0) *
                         jnp.maximum(ixmax - ixmin, 0.0))
                a2 = (jnp.maximum(ky1 - ky0, 0.0) *
                      jnp.maximum(kx1 - kx0, 0.0))
                union = a1 + a2 - inter
                safe = jnp.where(union > 0, union, 1.0)
                iou = jnp.where(union > 0, inter / safe, 0.0)
                return jnp.maximum(accmax, iou)

            accm = lax.fori_loop(0, nk, iou_body, zeros16)
            keep = jnp.max(accm) <= _IOU_THRESHOLD
            kf = jnp.where(keep, 1.0, 0.0).astype(jnp.float32)

            plsc.store_scatter(sw, [ivec + cb], negs16, mask=lane0)
            sv2 = plsc.load_gather(sw, [cb + si])
            plsc.store_scatter(m1, [splat(bstar)],
                               jnp.full((16,), jnp.max(sv2)), mask=lane0)
            mv2 = plsc.load_gather(m1, [mi])
            plsc.store_scatter(m2, [splat(jstar)],
                               jnp.full((16,), jnp.max(mv2)), mask=lane0)

            wl = splat(co + jnp.where(keep, k, 127))
            plsc.store_scatter(oS, [wl],
                               jnp.full((16,), gm) * kf, mask=lane0)
            plsc.store_scatter(oY0, [wl], by0 * kf, mask=lane0)
            plsc.store_scatter(oX0, [wl], bx0 * kf, mask=lane0)
            plsc.store_scatter(oY1, [wl], by1 * kf, mask=lane0)
            plsc.store_scatter(oX1, [wl], bx1 * kf, mask=lane0)

            return (k + keep.astype(jnp.int32), m2max())

        lax.while_loop(cond, body, (jnp.int32(0), m2max()))
        return 0

    lax.fori_loop(0, 5, class_body, 0)

    for r in range(5):
        sl = pl.ds(r * 128, 128)
        pltpu.sync_copy(oS.at[sl], outS.at[b, g, r])
        pltpu.sync_copy(oY0.at[sl], outY0.at[b, g, r])
        pltpu.sync_copy(oX0.at[sl], outX0.at[b, g, r])
        pltpu.sync_copy(oY1.at[sl], outY1.at[b, g, r])
        pltpu.sync_copy(oX1.at[sl], outX1.at[b, g, r])


def _sc_nms(scores_t, boxes_t, interpret=False):
    shp = jax.ShapeDtypeStruct((_B, 4, 5, 128), jnp.float32)
    return pl.kernel(
        _sc_nms_body,
        out_type=(shp, shp, shp, shp, shp),
        mesh=plsc.VectorSubcoreMesh(core_axis_name="c", subcore_axis_name="s"),
        compiler_params=pltpu.CompilerParams(use_tc_tiling_on_sc=False,
                                             needs_layout_passes=False),
        scratch_types=[
            pltpu.VMEM((5 * _NP,), jnp.float32),
            pltpu.VMEM((4 * _NP,), jnp.float32),
            pltpu.VMEM((_NB,), jnp.float32),
            pltpu.VMEM((32,), jnp.float32),
            pltpu.VMEM((640,), jnp.float32),
            pltpu.VMEM((640,), jnp.float32),
            pltpu.VMEM((640,), jnp.float32),
            pltpu.VMEM((640,), jnp.float32),
            pltpu.VMEM((640,), jnp.float32),
        ],
        interpret=interpret,
    )(scores_t, boxes_t)


def _merge_body(sS_ref, sY0_ref, sX0_ref, sY1_ref, sX1_ref, out_ref, merged):
    lane128 = lax.broadcasted_iota(jnp.int32, (_CP, 128), 1)
    row_iota = lax.broadcasted_iota(jnp.int32, (_CP, 1), 0)
    sS = sS_ref[0]
    cls_e = jnp.where(sS > 0.25, (row_iota + 1).astype(jnp.float32), 0.0)
    sY0 = sY0_ref[0]
    sX0 = sX0_ref[0]
    sY1 = sY1_ref[0]
    sX1 = sX1_ref[0]

    row8 = lax.broadcasted_iota(jnp.int32, (8, 256), 0)
    lane256 = lax.broadcasted_iota(jnp.int32, (8, 256), 1)
    merged[:] = jnp.zeros((8, 256), jnp.float32)

    def mstep(r, heads):
        hoh = lane128 == heads
        hs = jnp.sum(jnp.where(hoh, sS, 0.0), axis=1, keepdims=True)
        best = jnp.max(hs, axis=0, keepdims=True)
        flat = row_iota * _MAX_NMS + heads
        wflat = jnp.min(jnp.where(hs == best, flat, _BIGI), axis=0,
                        keepdims=True)
        wrow = flat == wflat
        woh = (wrow & hoh).astype(jnp.float32)
        vals = [jnp.sum(woh * a) for a in (cls_e, sS, sY0, sX0, sY1, sX1)]
        col = jnp.zeros((8, 256), jnp.float32)
        for k, v in enumerate(vals):
            col = col + jnp.where(row8 == k, v, 0.0)
        merged[:] = jnp.where(lane256 == r, col, merged[:])
        return heads + wrow.astype(jnp.int32)

    lax.fori_loop(0, _TOP_K, mstep, jnp.zeros((_CP, 1), jnp.int32))
    out_ref[0] = merged[:]


def _merge(sS, sY0, sX0, sY1, sX1, interpret=False):
    spec = pl.BlockSpec((1, _CP, 128), lambda b: (b, 0, 0))
    return pl.pallas_call(
        _merge_body,
        grid=(_B,),
        in_specs=[spec] * 5,
        out_specs=pl.BlockSpec((1, 8, 256), lambda b: (b, 0, 0)),
        out_shape=jax.ShapeDtypeStruct((_B, 8, 256), jnp.float32),
        scratch_shapes=[pltpu.VMEM((8, 256), jnp.float32)],
        interpret=interpret,
    )(sS, sY0, sX0, sY1, sX1)


def kernel(scores_pred, boxes_pred, _interpret=False):
    scores_t = jnp.transpose(scores_pred[:, :, 1:], (0, 2, 1))
    scores_t = jnp.pad(scores_t, ((0, 0), (0, 0), (0, _NP - _N)))
    scores_t = scores_t.reshape(_B, 4, 5, _NP)
    boxes_t = jnp.transpose(boxes_pred, (0, 2, 1))
    boxes_t = jnp.pad(boxes_t, ((0, 0), (0, 0), (0, _NP - _N)))
    outs = _sc_nms(scores_t, boxes_t, interpret=_interpret)
    sS, sY0, sX0, sY1, sX1 = (
        jnp.pad(o.reshape(_B, 20, 128), ((0, 0), (0, _CP - 20), (0, 0)))
        for o in outs)
    res = _merge(sS, sY0, sX0, sY1, sX1, interpret=_interpret)
    cls = res[:, 0, :_TOP_K]
    score = res[:, 1, :_TOP_K]
    top_scores = jnp.stack([cls, score], axis=-1)
    top_boxes = jnp.transpose(res[:, 2:6, :_TOP_K], (0, 2, 1))
    return top_scores, top_boxes


# speedup vs baseline: 16.3308x; 2.3543x over previous
import jax
import jax.numpy as jnp
from jax import lax
from jax.experimental import pallas as pl
from jax.experimental.pallas import tpu as pltpu
from jax.experimental.pallas import tpu_sc as plsc

_SCORE_THRESHOLD = 0.3
_IOU_THRESHOLD = 0.5
_TOP_K = 200
_MAX_NMS = 100
_B, _N, _C = 8, 5000, 21
_CP = 24
_NP = 5120
_NB = _NP // 16
_NEG = -1e30
_BIGI = 1 << 30


def _sc_nms_body(scores_hbm, boxes_hbm, outS, outY0, outX0, outY1, outX1,
                 sw, bx, m1, m2, oS, oY0, oX0, oY1, oX1):
    cid = lax.axis_index("c")
    sid = lax.axis_index("s")
    wid = cid * 16 + sid
    b = wid % 8
    g = wid // 8

    for r in range(5):
        pltpu.sync_copy(scores_hbm.at[b, g, r], sw.at[pl.ds(r * _NP, _NP)])
    for r in range(4):
        pltpu.sync_copy(boxes_hbm.at[b, r], bx.at[pl.ds(r * _NP, _NP)])

    iota = lax.iota(jnp.int32, 16)
    zeros16 = jnp.zeros((16,), jnp.float32)
    negs16 = jnp.full((16,), _NEG, jnp.float32)
    lane0 = iota == 0

    def splat(v):
        return jnp.full((16,), v, jnp.int32)

    def class_body(ci, _carry):
        cb = ci * _NP
        co = ci * 128
        for j in range(8):
            li = co + j * 16 + iota
            for ref in (oS, oY0, oX0, oY1, oX1):
                plsc.store_scatter(ref, [li], zeros16)

        def m1_body(jv, _):
            acc = negs16
            for kk in range(16):
                idx = cb + jv * 256 + iota * 16 + kk
                v = plsc.load_gather(sw, [idx])
                v = jnp.where(v > _SCORE_THRESHOLD, v, _NEG)
                plsc.store_scatter(sw, [idx], v)
                acc = jnp.maximum(acc, v)
            plsc.store_scatter(m1, [jv * 16 + iota], acc)
            return 0

        lax.fori_loop(0, _NB // 16, m1_body, 0)

        for jv2 in range(2):
            acc = negs16
            for kk in range(16):
                idxm = jnp.minimum((jv2 * 16 + iota) * 16 + kk, _NB - 1)
                acc = jnp.maximum(acc, plsc.load_gather(m1, [idxm]))
            if jv2 == 1:
                acc = jnp.where(jv2 * 16 + iota < 20, acc, _NEG)
            plsc.store_scatter(m2, [jv2 * 16 + iota], acc)

        def m2max():
            return jnp.max(jnp.maximum(m2[0:16], m2[16:32]))

        def cond(st):
            k, gm = st
            return (k < _MAX_NMS) & (gm > jnp.float32(-1e29))

        def body(st):
            k, gm = st
            v0 = m2[0:16]
            v1 = m2[16:32]
            c0 = jnp.where(v0 == gm, iota, _BIGI)
            c1 = jnp.where(v1 == gm, iota + 16, _BIGI)
            jstar = jnp.min(jnp.minimum(c0, c1))
            mi = jstar * 16 + iota
            mv = plsc.load_gather(m1, [mi])
            bstar = jnp.min(jnp.where(mv == gm, mi, _BIGI))
            si = bstar * 16 + iota
            sv = plsc.load_gather(sw, [cb + si])
            istar = jnp.min(jnp.where(sv == gm, si, _BIGI))
            ivec = splat(istar)
            by0 = plsc.load_gather(bx, [ivec])
            bx0 = plsc.load_gather(bx, [ivec + _NP])
            by1 = plsc.load_gather(bx, [ivec + 2 * _NP])
            bx1 = plsc.load_gather(bx, [ivec + 3 * _NP])
            a1 = jnp.maximum(by1 - by0, 0.0) * jnp.maximum(bx1 - bx0, 0.0)

            nk = (k + 15) // 16

            def iou_body(j, accmax):
                ki = co + j * 16 + iota
                ky0 = plsc.load_gather(oY0, [ki])
                kx0 = plsc.load_gather(oX0, [ki])
                ky1 = plsc.load_gather(oY1, [ki])
                kx1 = plsc.load_gather(oX1, [ki])
                iymin = jnp.maximum(by0, ky0)
                ixmin = jnp.maximum(bx0, kx0)
                iymax = jnp.minimum(by1, ky1)
                ixmax = jnp.minimum(bx1, kx1)
                inter = (jnp.maximum(iymax - iymin, 0.0) *
                         jnp.maximum(ixmax - ixmin, 0.0))
                a2 = (jnp.maximum(ky1 - ky0, 0.0) *
                      jnp.maximum(kx1 - kx0, 0.0))
                union = a1 + a2 - inter
                safe = jnp.where(union > 0, union, 1.0)
                iou = jnp.where(union > 0, inter / safe, 0.0)
                return jnp.maximum(accmax, iou)

            accm = lax.fori_loop(0, nk, iou_body, zeros16)
            keep = jnp.max(accm) <= _IOU_THRESHOLD
            kf = jnp.where(keep, 1.0, 0.0).astype(jnp.float32)

            plsc.store_scatter(sw, [ivec + cb], negs16, mask=lane0)
            sv2 = plsc.load_gather(sw, [cb + si])
            plsc.store_scatter(m1, [splat(bstar)],
                               jnp.full((16,), jnp.max(sv2)), mask=lane0)
            mv2 = plsc.load_gather(m1, [mi])
            plsc.store_scatter(m2, [splat(jstar)],
                               jnp.full((16,), jnp.max(mv2)), mask=lane0)

            wl = splat(co + jnp.where(keep, k, 127))
            plsc.store_scatter(oS, [wl],
                               jnp.full((16,), gm) * kf, mask=lane0)
            plsc.store_scatter(oY0, [wl], by0 * kf, mask=lane0)
            plsc.store_scatter(oX0, [wl], bx0 * kf, mask=lane0)
            plsc.store_scatter(oY1, [wl], by1 * kf, mask=lane0)
            plsc.store_scatter(oX1, [wl], bx1 * kf, mask=lane0)

            return (k + keep.astype(jnp.int32), m2max())

        lax.while_loop(cond, body, (jnp.int32(0), m2max()))
        return 0

    lax.fori_loop(0, 5, class_body, 0)

    for r in range(5):
        sl = pl.ds(r * 128, 128)
        pltpu.sync_copy(oS.at[sl], outS.at[b, g, r])
        pltpu.sync_copy(oY0.at[sl], outY0.at[b, g, r])
        pltpu.sync_copy(oX0.at[sl], outX0.at[b, g, r])
        pltpu.sync_copy(oY1.at[sl], outY1.at[b, g, r])
        pltpu.sync_copy(oX1.at[sl], outX1.at[b, g, r])


def _sc_nms(scores_t, boxes_t, interpret=False):
    shp = jax.ShapeDtypeStruct((_B, 4, 5, 128), jnp.float32)
    return pl.kernel(
        _sc_nms_body,
        out_type=(shp, shp, shp, shp, shp),
        mesh=plsc.VectorSubcoreMesh(core_axis_name="c", subcore_axis_name="s"),
        compiler_params=pltpu.CompilerParams(use_tc_tiling_on_sc=False,
                                             needs_layout_passes=False),
        scratch_types=[
            pltpu.VMEM((5 * _NP,), jnp.float32),
            pltpu.VMEM((4 * _NP,), jnp.float32),
            pltpu.VMEM((_NB,), jnp.float32),
            pltpu.VMEM((32,), jnp.float32),
            pltpu.VMEM((640,), jnp.float32),
            pltpu.VMEM((640,), jnp.float32),
            pltpu.VMEM((640,), jnp.float32),
            pltpu.VMEM((640,), jnp.float32),
            pltpu.VMEM((640,), jnp.float32),
        ],
        interpret=interpret,
    )(scores_t, boxes_t)


def _merge_body(sS_ref, sY0_ref, sX0_ref, sY1_ref, sX1_ref, out_ref, merged):
    lane128 = lax.broadcasted_iota(jnp.int32, (_B, _CP, 128), 2)
    row_iota = lax.broadcasted_iota(jnp.int32, (_B, _CP, 1), 1)
    sS = sS_ref[...]
    cls_e = jnp.where(sS > 0.25, (row_iota + 1).astype(jnp.float32), 0.0)
    sY0 = sY0_ref[...]
    sX0 = sX0_ref[...]
    sY1 = sY1_ref[...]
    sX1 = sX1_ref[...]

    row8 = lax.broadcasted_iota(jnp.int32, (_B, 8, 256), 1)
    lane256 = lax.broadcasted_iota(jnp.int32, (_B, 8, 256), 2)
    merged[...] = jnp.zeros((_B, 8, 256), jnp.float32)

    def mstep(r, heads):
        hoh = lane128 == heads
        hs = jnp.sum(jnp.where(hoh, sS, 0.0), axis=2, keepdims=True)
        best = jnp.max(hs, axis=1, keepdims=True)
        flat = row_iota * _MAX_NMS + heads
        wflat = jnp.min(jnp.where(hs == best, flat, _BIGI), axis=1,
                        keepdims=True)
        wrow = flat == wflat
        woh = (wrow & hoh).astype(jnp.float32)
        vals = [jnp.sum(jnp.sum(woh * a, axis=2, keepdims=True), axis=1,
                        keepdims=True)
                for a in (cls_e, sS, sY0, sX0, sY1, sX1)]
        col = jnp.zeros((_B, 8, 256), jnp.float32)
        for k, v in enumerate(vals):
            col = col + jnp.where(row8 == k, v, 0.0)
        merged[...] = jnp.where(lane256 == r, col, merged[...])
        return heads + wrow.astype(jnp.int32)

    lax.fori_loop(0, _TOP_K, mstep, jnp.zeros((_B, _CP, 1), jnp.int32))
    out_ref[...] = merged[...]


def _merge(sS, sY0, sX0, sY1, sX1, interpret=False):
    return pl.pallas_call(
        _merge_body,
        out_shape=jax.ShapeDtypeStruct((_B, 8, 256), jnp.float32),
        scratch_shapes=[pltpu.VMEM((_B, 8, 256), jnp.float32)],
        interpret=interpret,
    )(sS, sY0, sX0, sY1, sX1)


def kernel(scores_pred, boxes_pred, _interpret=False):
    scores_t = jnp.transpose(scores_pred[:, :, 1:], (0, 2, 1))
    scores_t = jnp.pad(scores_t, ((0, 0), (0, 0), (0, _NP - _N)))
    scores_t = scores_t.reshape(_B, 4, 5, _NP)
    boxes_t = jnp.transpose(boxes_pred, (0, 2, 1))
    boxes_t = jnp.pad(boxes_t, ((0, 0), (0, 0), (0, _NP - _N)))
    outs = _sc_nms(scores_t, boxes_t, interpret=_interpret)
    sS, sY0, sX0, sY1, sX1 = (
        jnp.pad(o.reshape(_B, 20, 128), ((0, 0), (0, _CP - 20), (0, 0)))
        for o in outs)
    res = _merge(sS, sY0, sX0, sY1, sX1, interpret=_interpret)
    cls = res[:, 0, :_TOP_K]
    score = res[:, 1, :_TOP_K]
    top_scores = jnp.stack([cls, score], axis=-1)
    top_boxes = jnp.transpose(res[:, 2:6, :_TOP_K], (0, 2, 1))
    return top_scores, top_boxes


# speedup vs baseline: 23.7137x; 1.4521x over previous
import jax
import jax.numpy as jnp
from jax import lax
from jax.experimental import pallas as pl
from jax.experimental.pallas import tpu as pltpu
from jax.experimental.pallas import tpu_sc as plsc

_SCORE_THRESHOLD = 0.3
_IOU_THRESHOLD = 0.5
_TOP_K = 200
_MAX_NMS = 100
_B, _N, _C = 8, 5000, 21
_CP = 24
_NP = 5120
_NB = _NP // 16
_NEG = -1e30
_BIGI = 1 << 30


def _sc_nms_body(scores_hbm, boxes_hbm, out_hbm,
                 sw, bx, m1, m2, oS, oY0, oX0, oY1, oX1, oM, shared):
    cid = lax.axis_index("c")
    sid = lax.axis_index("s")
    b = cid * 4 + sid // 4
    g = sid % 4

    for r in range(5):
        pltpu.sync_copy(scores_hbm.at[b, g, r], sw.at[pl.ds(r * _NP, _NP)])
    for r in range(4):
        pltpu.sync_copy(boxes_hbm.at[b, r], bx.at[pl.ds(r * _NP, _NP)])

    iota = lax.iota(jnp.int32, 16)
    zeros16 = jnp.zeros((16,), jnp.float32)
    negs16 = jnp.full((16,), _NEG, jnp.float32)
    lane0 = iota == 0

    def splat(v):
        return jnp.full((16,), v, jnp.int32)

    def class_body(ci, _carry):
        cb = ci * _NP
        co = ci * 128
        for j in range(8):
            li = co + j * 16 + iota
            for ref in (oS, oY0, oX0, oY1, oX1):
                plsc.store_scatter(ref, [li], zeros16)

        def m1_body(jv, _):
            acc = negs16
            for kk in range(16):
                idx = cb + jv * 256 + iota * 16 + kk
                v = plsc.load_gather(sw, [idx])
                v = jnp.where(v > _SCORE_THRESHOLD, v, _NEG)
                plsc.store_scatter(sw, [idx], v)
                acc = jnp.maximum(acc, v)
            plsc.store_scatter(m1, [jv * 16 + iota], acc)
            return 0

        lax.fori_loop(0, _NB // 16, m1_body, 0)

        for jv2 in range(2):
            acc = negs16
            for kk in range(16):
                idxm = jnp.minimum((jv2 * 16 + iota) * 16 + kk, _NB - 1)
                acc = jnp.maximum(acc, plsc.load_gather(m1, [idxm]))
            if jv2 == 1:
                acc = jnp.where(jv2 * 16 + iota < 20, acc, _NEG)
            plsc.store_scatter(m2, [jv2 * 16 + iota], acc)

        def m2max():
            return jnp.max(jnp.maximum(m2[0:16], m2[16:32]))

        def cond(st):
            k, gm = st
            return (k < _MAX_NMS) & (gm > jnp.float32(-1e29))

        def body(st):
            k, gm = st
            v0 = m2[0:16]
            v1 = m2[16:32]
            c0 = jnp.where(v0 == gm, iota, _BIGI)
            c1 = jnp.where(v1 == gm, iota + 16, _BIGI)
            jstar = jnp.min(jnp.minimum(c0, c1))
            mi = jstar * 16 + iota
            mv = plsc.load_gather(m1, [mi])
            bstar = jnp.min(jnp.where(mv == gm, mi, _BIGI))
            si = bstar * 16 + iota
            sv = plsc.load_gather(sw, [cb + si])
            istar = jnp.min(jnp.where(sv == gm, si, _BIGI))
            ivec = splat(istar)
            by0 = plsc.load_gather(bx, [ivec])
            bx0 = plsc.load_gather(bx, [ivec + _NP])
            by1 = plsc.load_gather(bx, [ivec + 2 * _NP])
            bx1 = plsc.load_gather(bx, [ivec + 3 * _NP])
            a1 = jnp.maximum(by1 - by0, 0.0) * jnp.maximum(bx1 - bx0, 0.0)

            nk = (k + 15) // 16

            def iou_body(j, accmax):
                ki = co + j * 16 + iota
                ky0 = plsc.load_gather(oY0, [ki])
                kx0 = plsc.load_gather(oX0, [ki])
                ky1 = plsc.load_gather(oY1, [ki])
                kx1 = plsc.load_gather(oX1, [ki])
                iymin = jnp.maximum(by0, ky0)
                ixmin = jnp.maximum(bx0, kx0)
                iymax = jnp.minimum(by1, ky1)
                ixmax = jnp.minimum(bx1, kx1)
                inter = (jnp.maximum(iymax - iymin, 0.0) *
                         jnp.maximum(ixmax - ixmin, 0.0))
                a2 = (jnp.maximum(ky1 - ky0, 0.0) *
                      jnp.maximum(kx1 - kx0, 0.0))
                union = a1 + a2 - inter
                safe = jnp.where(union > 0, union, 1.0)
                iou = jnp.where(union > 0, inter / safe, 0.0)
                return jnp.maximum(accmax, iou)

            accm = lax.fori_loop(0, nk, iou_body, zeros16)
            keep = jnp.max(accm) <= _IOU_THRESHOLD
            kf = jnp.where(keep, 1.0, 0.0).astype(jnp.float32)

            plsc.store_scatter(sw, [ivec + cb], negs16, mask=lane0)
            sv2 = plsc.load_gather(sw, [cb + si])
            plsc.store_scatter(m1, [splat(bstar)],
                               jnp.full((16,), jnp.max(sv2)), mask=lane0)
            mv2 = plsc.load_gather(m1, [mi])
            plsc.store_scatter(m2, [splat(jstar)],
                               jnp.full((16,), jnp.max(mv2)), mask=lane0)

            wl = splat(co + jnp.where(keep, k, 127))
            plsc.store_scatter(oS, [wl],
                               jnp.full((16,), gm) * kf, mask=lane0)
            plsc.store_scatter(oY0, [wl], by0 * kf, mask=lane0)
            plsc.store_scatter(oX0, [wl], bx0 * kf, mask=lane0)
            plsc.store_scatter(oY1, [wl], by1 * kf, mask=lane0)
            plsc.store_scatter(oX1, [wl], bx1 * kf, mask=lane0)

            return (k + keep.astype(jnp.int32), m2max())

        lax.while_loop(cond, body, (jnp.int32(0), m2max()))
        return 0

    lax.fori_loop(0, 5, class_body, 0)

    pltpu.sync_copy(oS, shared.at[sid, pl.ds(0, 640)])
    pltpu.sync_copy(oY0, shared.at[sid, pl.ds(640, 640)])
    pltpu.sync_copy(oX0, shared.at[sid, pl.ds(1280, 640)])
    pltpu.sync_copy(oY1, shared.at[sid, pl.ds(1920, 640)])
    pltpu.sync_copy(oX1, shared.at[sid, pl.ds(2560, 640)])
    plsc.subcore_barrier()

    @pl.when(sid % 4 == 0)
    def _():
        for j in range(4):
            pltpu.sync_copy(shared.at[sid + j], sw.at[pl.ds(j * 3200, 3200)])
        cc0 = iota
        cc1 = iota + 16
        base0 = (cc0 // 5) * 3200 + (cc0 % 5) * 128
        base1 = (cc1 // 5) * 3200 + (cc1 % 5) * 128

        def mstep(r, heads):
            h0, h1 = heads
            g0v = plsc.load_gather(sw, [base0 + jnp.minimum(h0, 127)])
            hs0 = jnp.where(h0 < 128, g0v, 0.0)
            g1v = plsc.load_gather(sw, [base1 + jnp.minimum(h1, 127)])
            hs1 = jnp.where((h1 < 128) & (cc1 < 20), g1v, _NEG)
            gmax = jnp.max(jnp.maximum(hs0, hs1))
            key0 = jnp.where(hs0 == gmax, cc0 * 256 + h0, _BIGI)
            key1 = jnp.where(hs1 == gmax, cc1 * 256 + h1, _BIGI)
            wkey = jnp.min(jnp.minimum(key0, key1))
            wcc = wkey // 256
            wh = wkey % 256
            basew = ((wcc // 5) * 3200 + (wcc % 5) * 128 +
                     jnp.minimum(wh, 127))
            cls_val = jnp.where(gmax > 0.25,
                                (wcc + 1).astype(jnp.float32), 0.0)
            plsc.store_scatter(oM, [splat(r)],
                               jnp.full((16,), cls_val), mask=lane0)
            plsc.store_scatter(oM, [splat(256 + r)],
                               jnp.full((16,), gmax), mask=lane0)
            for a in range(1, 5):
                va = plsc.load_gather(sw, [splat(basew + a * 640)])
                plsc.store_scatter(oM, [splat((a + 1) * 256 + r)], va,
                                   mask=lane0)
            h0n = h0 + (cc0 == wcc).astype(jnp.int32)
            h1n = h1 + (cc1 == wcc).astype(jnp.int32)
            return (h0n, h1n)

        zi = jnp.zeros((16,), jnp.int32)
        lax.fori_loop(0, _TOP_K, mstep, (zi, zi))
        for a in range(6):
            pltpu.sync_copy(oM.at[pl.ds(a * 256, 256)], out_hbm.at[b, a])


def _sc_nms(scores_t, boxes_t, interpret=False):
    return pl.kernel(
        _sc_nms_body,
        out_type=jax.ShapeDtypeStruct((_B, 6, 256), jnp.float32),
        mesh=plsc.VectorSubcoreMesh(core_axis_name="c", subcore_axis_name="s"),
        compiler_params=pltpu.CompilerParams(use_tc_tiling_on_sc=False,
                                             needs_layout_passes=False),
        scratch_types=[
            pltpu.VMEM((5 * _NP,), jnp.float32),
            pltpu.VMEM((4 * _NP,), jnp.float32),
            pltpu.VMEM((_NB,), jnp.float32),
            pltpu.VMEM((32,), jnp.float32),
            pltpu.VMEM((640,), jnp.float32),
            pltpu.VMEM((640,), jnp.float32),
            pltpu.VMEM((640,), jnp.float32),
            pltpu.VMEM((640,), jnp.float32),
            pltpu.VMEM((640,), jnp.float32),
            pltpu.VMEM((1536,), jnp.float32),
            pltpu.VMEM_SHARED((16, 3200), jnp.float32),
        ],
        interpret=interpret,
    )(scores_t, boxes_t)


def _merge_body(sS_ref, sY0_ref, sX0_ref, sY1_ref, sX1_ref, out_ref, merged):
    lane128 = lax.broadcasted_iota(jnp.int32, (_B, _CP, 128), 2)
    row_iota = lax.broadcasted_iota(jnp.int32, (_B, _CP, 1), 1)
    sS = sS_ref[...]
    cls_e = jnp.where(sS > 0.25, (row_iota + 1).astype(jnp.float32), 0.0)
    sY0 = sY0_ref[...]
    sX0 = sX0_ref[...]
    sY1 = sY1_ref[...]
    sX1 = sX1_ref[...]

    row8 = lax.broadcasted_iota(jnp.int32, (_B, 8, 256), 1)
    lane256 = lax.broadcasted_iota(jnp.int32, (_B, 8, 256), 2)
    merged[...] = jnp.zeros((_B, 8, 256), jnp.float32)

    def mstep(r, heads):
        hoh = lane128 == heads
        hs = jnp.sum(jnp.where(hoh, sS, 0.0), axis=2, keepdims=True)
        best = jnp.max(hs, axis=1, keepdims=True)
        flat = row_iota * _MAX_NMS + heads
        wflat = jnp.min(jnp.where(hs == best, flat, _BIGI), axis=1,
                        keepdims=True)
        wrow = flat == wflat
        woh = (wrow & hoh).astype(jnp.float32)
        vals = [jnp.sum(jnp.sum(woh * a, axis=2, keepdims=True), axis=1,
                        keepdims=True)
                for a in (cls_e, sS, sY0, sX0, sY1, sX1)]
        col = jnp.zeros((_B, 8, 256), jnp.float32)
        for k, v in enumerate(vals):
            col = col + jnp.where(row8 == k, v, 0.0)
        merged[...] = jnp.where(lane256 == r, col, merged[...])
        return heads + wrow.astype(jnp.int32)

    lax.fori_loop(0, _TOP_K, mstep, jnp.zeros((_B, _CP, 1), jnp.int32))
    out_ref[...] = merged[...]


def _merge(sS, sY0, sX0, sY1, sX1, interpret=False):
    return pl.pallas_call(
        _merge_body,
        out_shape=jax.ShapeDtypeStruct((_B, 8, 256), jnp.float32),
        scratch_shapes=[pltpu.VMEM((_B, 8, 256), jnp.float32)],
        interpret=interpret,
    )(sS, sY0, sX0, sY1, sX1)


def kernel(scores_pred, boxes_pred, _interpret=False):
    scores_t = jnp.transpose(scores_pred[:, :, 1:], (0, 2, 1))
    scores_t = jnp.pad(scores_t, ((0, 0), (0, 0), (0, _NP - _N)))
    scores_t = scores_t.reshape(_B, 4, 5, _NP)
    boxes_t = jnp.transpose(boxes_pred, (0, 2, 1))
    boxes_t = jnp.pad(boxes_t, ((0, 0), (0, 0), (0, _NP - _N)))
    res = _sc_nms(scores_t, boxes_t, interpret=_interpret)
    cls = res[:, 0, :_TOP_K]
    score = res[:, 1, :_TOP_K]
    top_scores = jnp.stack([cls, score], axis=-1)
    top_boxes = jnp.transpose(res[:, 2:6, :_TOP_K], (0, 2, 1))
    return top_scores, top_boxes


# speedup vs baseline: 24.9422x; 1.0518x over previous
import jax
import jax.numpy as jnp
from jax import lax
from jax.experimental import pallas as pl
from jax.experimental.pallas import tpu as pltpu
from jax.experimental.pallas import tpu_sc as plsc

_SCORE_THRESHOLD = 0.3
_IOU_THRESHOLD = 0.5
_TOP_K = 200
_MAX_NMS = 100
_B, _N, _C = 8, 5000, 21
_CP = 24
_NP = 5120
_NB = _NP // 16
_NEG = -1e30
_BIGI = 1 << 30


def _sc_nms_body(scores_hbm, boxes_hbm, out_hbm,
                 sw, bx, m1, oS, oY0, oX0, oY1, oX1, oM, shared):
    cid = lax.axis_index("c")
    sid = lax.axis_index("s")
    b = cid * 4 + sid // 4
    g = sid % 4

    for r in range(5):
        pltpu.sync_copy(scores_hbm.at[b, g, r], sw.at[pl.ds(r * _NP, _NP)])
    for r in range(4):
        pltpu.sync_copy(boxes_hbm.at[b, r], bx.at[pl.ds(r * _NP, _NP)])

    iota = lax.iota(jnp.int32, 16)
    zeros16 = jnp.zeros((16,), jnp.float32)
    negs16 = jnp.full((16,), _NEG, jnp.float32)
    lane0 = iota == 0

    def splat(v):
        return jnp.full((16,), v, jnp.int32)

    def class_body(ci, _carry):
        cb = ci * _NP
        co = ci * 128
        for j in range(8):
            li = co + j * 16 + iota
            for ref in (oS, oY0, oX0, oY1, oX1):
                plsc.store_scatter(ref, [li], zeros16)

        def m1_body(jv, _):
            acc = negs16
            for kk in range(16):
                idx = cb + jv * 256 + iota * 16 + kk
                v = plsc.load_gather(sw, [idx])
                v = jnp.where(v > _SCORE_THRESHOLD, v, _NEG)
                plsc.store_scatter(sw, [idx], v)
                acc = jnp.maximum(acc, v)
            plsc.store_scatter(m1, [jv * 16 + iota], acc)
            return 0

        lax.fori_loop(0, _NB // 16, m1_body, 0)

        m2a = negs16
        for kk in range(16):
            m2a = jnp.maximum(m2a, plsc.load_gather(m1, [iota * 16 + kk]))
        m2b = negs16
        for kk in range(16):
            idxm = jnp.minimum((16 + iota) * 16 + kk, _NB - 1)
            m2b = jnp.maximum(m2b, plsc.load_gather(m1, [idxm]))
        m2b = jnp.where(iota < 4, m2b, _NEG)

        def cond(st):
            k, gm, _v0, _v1 = st
            return (k < _MAX_NMS) & (gm > jnp.float32(-1e29))

        def body(st):
            k, gm, v0, v1 = st
            c0 = jnp.where(v0 == gm, iota, _BIGI)
            c1 = jnp.where(v1 == gm, iota + 16, _BIGI)
            jstar = jnp.min(jnp.minimum(c0, c1))
            mi = jstar * 16 + iota
            mv = plsc.load_gather(m1, [mi])
            bstar = jnp.min(jnp.where(mv == gm, mi, _BIGI))
            si = bstar * 16 + iota
            sv = plsc.load_gather(sw, [cb + si])
            istar = jnp.min(jnp.where(sv == gm, si, _BIGI))
            ivec = splat(istar)
            by0 = plsc.load_gather(bx, [ivec])
            bx0 = plsc.load_gather(bx, [ivec + _NP])
            by1 = plsc.load_gather(bx, [ivec + 2 * _NP])
            bx1 = plsc.load_gather(bx, [ivec + 3 * _NP])
            a1 = jnp.maximum(by1 - by0, 0.0) * jnp.maximum(bx1 - bx0, 0.0)

            nk = (k + 15) // 16

            def iou_body(j, accmax):
                ki = co + j * 16 + iota
                ky0 = plsc.load_gather(oY0, [ki])
                kx0 = plsc.load_gather(oX0, [ki])
                ky1 = plsc.load_gather(oY1, [ki])
                kx1 = plsc.load_gather(oX1, [ki])
                iymin = jnp.maximum(by0, ky0)
                ixmin = jnp.maximum(bx0, kx0)
                iymax = jnp.minimum(by1, ky1)
                ixmax = jnp.minimum(bx1, kx1)
                inter = (jnp.maximum(iymax - iymin, 0.0) *
                         jnp.maximum(ixmax - ixmin, 0.0))
                a2 = (jnp.maximum(ky1 - ky0, 0.0) *
                      jnp.maximum(kx1 - kx0, 0.0))
                union = a1 + a2 - inter
                safe = jnp.where(union > 0, union, 1.0)
                iou = jnp.where(union > 0, inter / safe, 0.0)
                return jnp.maximum(accmax, iou)

            accm = lax.fori_loop(0, nk, iou_body, zeros16)
            keep = jnp.max(accm) <= _IOU_THRESHOLD
            kf = jnp.where(keep, 1.0, 0.0).astype(jnp.float32)

            plsc.store_scatter(sw, [ivec + cb], negs16, mask=lane0)
            nb = jnp.max(jnp.where(si == istar, _NEG, sv))
            plsc.store_scatter(m1, [splat(bstar)], jnp.full((16,), nb),
                               mask=lane0)
            nm2 = jnp.max(jnp.where(mi == bstar, nb, mv))
            v0n = jnp.where(iota == jstar, nm2, v0)
            v1n = jnp.where(iota + 16 == jstar, nm2, v1)

            wl = splat(co + jnp.where(keep, k, 127))
            plsc.store_scatter(oS, [wl],
                               jnp.full((16,), gm) * kf, mask=lane0)
            plsc.store_scatter(oY0, [wl], by0 * kf, mask=lane0)
            plsc.store_scatter(oX0, [wl], bx0 * kf, mask=lane0)
            plsc.store_scatter(oY1, [wl], by1 * kf, mask=lane0)
            plsc.store_scatter(oX1, [wl], bx1 * kf, mask=lane0)

            gm2 = jnp.max(jnp.maximum(v0n, v1n))
            return (k + keep.astype(jnp.int32), gm2, v0n, v1n)

        gm0 = jnp.max(jnp.maximum(m2a, m2b))
        lax.while_loop(cond, body, (jnp.int32(0), gm0, m2a, m2b))
        return 0

    lax.fori_loop(0, 5, class_body, 0)

    pltpu.sync_copy(oS, shared.at[sid, pl.ds(0, 640)])
    pltpu.sync_copy(oY0, shared.at[sid, pl.ds(640, 640)])
    pltpu.sync_copy(oX0, shared.at[sid, pl.ds(1280, 640)])
    pltpu.sync_copy(oY1, shared.at[sid, pl.ds(1920, 640)])
    pltpu.sync_copy(oX1, shared.at[sid, pl.ds(2560, 640)])
    plsc.subcore_barrier()

    @pl.when(sid % 4 == 0)
    def _():
        for j in range(4):
            pltpu.sync_copy(shared.at[sid + j], sw.at[pl.ds(j * 3200, 3200)])
        cc0 = iota
        cc1 = iota + 16
        base0 = (cc0 // 5) * 3200 + (cc0 % 5) * 128
        base1 = (cc1 // 5) * 3200 + (cc1 % 5) * 128

        def mstep(r, heads):
            h0, h1 = heads
            g0v = plsc.load_gather(sw, [base0 + jnp.minimum(h0, 127)])
            hs0 = jnp.where(h0 < 128, g0v, 0.0)
            g1v = plsc.load_gather(sw, [base1 + jnp.minimum(h1, 127)])
            hs1 = jnp.where((h1 < 128) & (cc1 < 20), g1v, _NEG)
            gmax = jnp.max(jnp.maximum(hs0, hs1))
            key0 = jnp.where(hs0 == gmax, cc0 * 256 + h0, _BIGI)
            key1 = jnp.where(hs1 == gmax, cc1 * 256 + h1, _BIGI)
            wkey = jnp.min(jnp.minimum(key0, key1))
            wcc = wkey // 256
            wh = wkey % 256
            basew = ((wcc // 5) * 3200 + (wcc % 5) * 128 +
                     jnp.minimum(wh, 127))
            cls_val = jnp.where(gmax > 0.25,
                                (wcc + 1).astype(jnp.float32), 0.0)
            plsc.store_scatter(oM, [splat(r)],
                               jnp.full((16,), cls_val), mask=lane0)
            plsc.store_scatter(oM, [splat(256 + r)],
                               jnp.full((16,), gmax), mask=lane0)
            for a in range(1, 5):
                va = plsc.load_gather(sw, [splat(basew + a * 640)])
                plsc.store_scatter(oM, [splat((a + 1) * 256 + r)], va,
                                   mask=lane0)
            h0n = h0 + (cc0 == wcc).astype(jnp.int32)
            h1n = h1 + (cc1 == wcc).astype(jnp.int32)
            return (h0n, h1n)

        zi = jnp.zeros((16,), jnp.int32)
        lax.fori_loop(0, _TOP_K, mstep, (zi, zi))
        for a in range(6):
            pltpu.sync_copy(oM.at[pl.ds(a * 256, 256)], out_hbm.at[b, a])


def _sc_nms(scores_t, boxes_t, interpret=False):
    return pl.kernel(
        _sc_nms_body,
        out_type=jax.ShapeDtypeStruct((_B, 6, 256), jnp.float32),
        mesh=plsc.VectorSubcoreMesh(core_axis_name="c", subcore_axis_name="s"),
        compiler_params=pltpu.CompilerParams(use_tc_tiling_on_sc=False,
                                             needs_layout_passes=False),
        scratch_types=[
            pltpu.VMEM((5 * _NP,), jnp.float32),
            pltpu.VMEM((4 * _NP,), jnp.float32),
            pltpu.VMEM((_NB,), jnp.float32),
            pltpu.VMEM((640,), jnp.float32),
            pltpu.VMEM((640,), jnp.float32),
            pltpu.VMEM((640,), jnp.float32),
            pltpu.VMEM((640,), jnp.float32),
            pltpu.VMEM((640,), jnp.float32),
            pltpu.VMEM((1536,), jnp.float32),
            pltpu.VMEM_SHARED((16, 3200), jnp.float32),
        ],
        interpret=interpret,
    )(scores_t, boxes_t)


def _merge_body(sS_ref, sY0_ref, sX0_ref, sY1_ref, sX1_ref, out_ref, merged):
    lane128 = lax.broadcasted_iota(jnp.int32, (_B, _CP, 128), 2)
    row_iota = lax.broadcasted_iota(jnp.int32, (_B, _CP, 1), 1)
    sS = sS_ref[...]
    cls_e = jnp.where(sS > 0.25, (row_iota + 1).astype(jnp.float32), 0.0)
    sY0 = sY0_ref[...]
    sX0 = sX0_ref[...]
    sY1 = sY1_ref[...]
    sX1 = sX1_ref[...]

    row8 = lax.broadcasted_iota(jnp.int32, (_B, 8, 256), 1)
    lane256 = lax.broadcasted_iota(jnp.int32, (_B, 8, 256), 2)
    merged[...] = jnp.zeros((_B, 8, 256), jnp.float32)

    def mstep(r, heads):
        hoh = lane128 == heads
        hs = jnp.sum(jnp.where(hoh, sS, 0.0), axis=2, keepdims=True)
        best = jnp.max(hs, axis=1, keepdims=True)
        flat = row_iota * _MAX_NMS + heads
        wflat = jnp.min(jnp.where(hs == best, flat, _BIGI), axis=1,
                        keepdims=True)
        wrow = flat == wflat
        woh = (wrow & hoh).astype(jnp.float32)
        vals = [jnp.sum(jnp.sum(woh * a, axis=2, keepdims=True), axis=1,
                        keepdims=True)
                for a in (cls_e, sS, sY0, sX0, sY1, sX1)]
        col = jnp.zeros((_B, 8, 256), jnp.float32)
        for k, v in enumerate(vals):
            col = col + jnp.where(row8 == k, v, 0.0)
        merged[...] = jnp.where(lane256 == r, col, merged[...])
        return heads + wrow.astype(jnp.int32)

    lax.fori_loop(0, _TOP_K, mstep, jnp.zeros((_B, _CP, 1), jnp.int32))
    out_ref[...] = merged[...]


def _merge(sS, sY0, sX0, sY1, sX1, interpret=False):
    return pl.pallas_call(
        _merge_body,
        out_shape=jax.ShapeDtypeStruct((_B, 8, 256), jnp.float32),
        scratch_shapes=[pltpu.VMEM((_B, 8, 256), jnp.float32)],
        interpret=interpret,
    )(sS, sY0, sX0, sY1, sX1)


def kernel(scores_pred, boxes_pred, _interpret=False):
    scores_t = jnp.transpose(scores_pred[:, :, 1:], (0, 2, 1))
    scores_t = jnp.pad(scores_t, ((0, 0), (0, 0), (0, _NP - _N)))
    scores_t = scores_t.reshape(_B, 4, 5, _NP)
    boxes_t = jnp.transpose(boxes_pred, (0, 2, 1))
    boxes_t = jnp.pad(boxes_t, ((0, 0), (0, 0), (0, _NP - _N)))
    res = _sc_nms(scores_t, boxes_t, interpret=_interpret)
    cls = res[:, 0, :_TOP_K]
    score = res[:, 1, :_TOP_K]
    top_scores = jnp.stack([cls, score], axis=-1)
    top_boxes = jnp.transpose(res[:, 2:6, :_TOP_K], (0, 2, 1))
    return top_scores, top_boxes


# speedup vs baseline: 27.2315x; 1.0918x over previous
import jax
import jax.numpy as jnp
from jax import lax
from jax.experimental import pallas as pl
from jax.experimental.pallas import tpu as pltpu
from jax.experimental.pallas import tpu_sc as plsc

_SCORE_THRESHOLD = 0.3
_IOU_THRESHOLD = 0.5
_TOP_K = 200
_MAX_NMS = 100
_B, _N, _C = 8, 5000, 21
_CP = 24
_NP = 5120
_NB = _NP // 16
_NEG = -1e30
_BIGI = 1 << 30


def _sc_nms_body(scores_hbm, boxes_hbm, out_hbm,
                 sw, bx, m1, oS, oY0, oX0, oY1, oX1, oM, shared):
    cid = lax.axis_index("c")
    sid = lax.axis_index("s")
    b = cid * 4 + sid // 4
    g = sid % 4

    for r in range(5):
        pltpu.sync_copy(scores_hbm.at[b, g, r], sw.at[pl.ds(r * _NP, _NP)])
    for r in range(4):
        pltpu.sync_copy(boxes_hbm.at[b, r], bx.at[pl.ds(r * _NP, _NP)])

    iota = lax.iota(jnp.int32, 16)
    zeros16 = jnp.zeros((16,), jnp.float32)
    negs16 = jnp.full((16,), _NEG, jnp.float32)
    lane0 = iota == 0

    def splat(v):
        return jnp.full((16,), v, jnp.int32)

    def class_body(ci, _carry):
        cb = ci * _NP
        co = ci * 128
        for j in range(8):
            li = co + j * 16 + iota
            for ref in (oS, oY0, oX0, oY1, oX1):
                plsc.store_scatter(ref, [li], zeros16)

        def m1_body(jv, _):
            acc = negs16
            for kk in range(16):
                idx = cb + jv * 256 + iota * 16 + kk
                acc = jnp.maximum(acc, plsc.load_gather(sw, [idx]))
            plsc.store_scatter(m1, [jv * 16 + iota], acc)
            return 0

        lax.fori_loop(0, _NB // 16, m1_body, 0)

        m2a = negs16
        for kk in range(16):
            m2a = jnp.maximum(m2a, plsc.load_gather(m1, [iota * 16 + kk]))
        m2b = negs16
        for kk in range(16):
            idxm = jnp.minimum((16 + iota) * 16 + kk, _NB - 1)
            m2b = jnp.maximum(m2b, plsc.load_gather(m1, [idxm]))
        m2b = jnp.where(iota < 4, m2b, _NEG)

        def cond(st):
            k, gm, _v0, _v1 = st
            return (k < _MAX_NMS) & (gm > jnp.float32(_SCORE_THRESHOLD))

        def body(st):
            k, gm, v0, v1 = st
            c0 = jnp.where(v0 == gm, iota, _BIGI)
            c1 = jnp.where(v1 == gm, iota + 16, _BIGI)
            jstar = jnp.min(jnp.minimum(c0, c1))
            mi = jstar * 16 + iota
            mv = plsc.load_gather(m1, [mi])
            bstar = jnp.min(jnp.where(mv == gm, mi, _BIGI))
            si = bstar * 16 + iota
            sv = plsc.load_gather(sw, [cb + si])
            istar = jnp.min(jnp.where(sv == gm, si, _BIGI))
            ivec = splat(istar)
            by0 = plsc.load_gather(bx, [ivec])
            bx0 = plsc.load_gather(bx, [ivec + _NP])
            by1 = plsc.load_gather(bx, [ivec + 2 * _NP])
            bx1 = plsc.load_gather(bx, [ivec + 3 * _NP])
            a1 = jnp.maximum(by1 - by0, 0.0) * jnp.maximum(bx1 - bx0, 0.0)

            nk = (k + 31) // 32

            def iou16(ki):
                ky0 = plsc.load_gather(oY0, [ki])
                kx0 = plsc.load_gather(oX0, [ki])
                ky1 = plsc.load_gather(oY1, [ki])
                kx1 = plsc.load_gather(oX1, [ki])
                iymin = jnp.maximum(by0, ky0)
                ixmin = jnp.maximum(bx0, kx0)
                iymax = jnp.minimum(by1, ky1)
                ixmax = jnp.minimum(bx1, kx1)
                inter = (jnp.maximum(iymax - iymin, 0.0) *
                         jnp.maximum(ixmax - ixmin, 0.0))
                a2 = (jnp.maximum(ky1 - ky0, 0.0) *
                      jnp.maximum(kx1 - kx0, 0.0))
                union = a1 + a2 - inter
                safe = jnp.where(union > 0, union, 1.0)
                return jnp.where(union > 0, inter / safe, 0.0)

            def iou_body(j, accmax):
                ki = co + j * 32 + iota
                return jnp.maximum(accmax,
                                   jnp.maximum(iou16(ki), iou16(ki + 16)))

            accm = lax.fori_loop(0, nk, iou_body, zeros16)
            keep = jnp.max(accm) <= _IOU_THRESHOLD
            kf = jnp.where(keep, 1.0, 0.0).astype(jnp.float32)

            plsc.store_scatter(sw, [ivec + cb], negs16, mask=lane0)
            nb = jnp.max(jnp.where(si == istar, _NEG, sv))
            plsc.store_scatter(m1, [splat(bstar)], jnp.full((16,), nb),
                               mask=lane0)
            nm2 = jnp.max(jnp.where(mi == bstar, nb, mv))
            v0n = jnp.where(iota == jstar, nm2, v0)
            v1n = jnp.where(iota + 16 == jstar, nm2, v1)

            wl = splat(co + jnp.where(keep, k, 127))
            plsc.store_scatter(oS, [wl],
                               jnp.full((16,), gm) * kf, mask=lane0)
            plsc.store_scatter(oY0, [wl], by0 * kf, mask=lane0)
            plsc.store_scatter(oX0, [wl], bx0 * kf, mask=lane0)
            plsc.store_scatter(oY1, [wl], by1 * kf, mask=lane0)
            plsc.store_scatter(oX1, [wl], bx1 * kf, mask=lane0)

            gm2 = jnp.max(jnp.maximum(v0n, v1n))
            return (k + keep.astype(jnp.int32), gm2, v0n, v1n)

        gm0 = jnp.max(jnp.maximum(m2a, m2b))
        lax.while_loop(cond, body, (jnp.int32(0), gm0, m2a, m2b))
        return 0

    lax.fori_loop(0, 5, class_body, 0)

    pltpu.sync_copy(oS, shared.at[sid, pl.ds(0, 640)])
    pltpu.sync_copy(oY0, shared.at[sid, pl.ds(640, 640)])
    pltpu.sync_copy(oX0, shared.at[sid, pl.ds(1280, 640)])
    pltpu.sync_copy(oY1, shared.at[sid, pl.ds(1920, 640)])
    pltpu.sync_copy(oX1, shared.at[sid, pl.ds(2560, 640)])
    plsc.subcore_barrier()

    @pl.when(sid % 4 == 0)
    def _():
        for j in range(4):
            pltpu.sync_copy(shared.at[sid + j], sw.at[pl.ds(j * 3200, 3200)])
        cc0 = iota
        cc1 = iota + 16
        base0 = (cc0 // 5) * 3200 + (cc0 % 5) * 128
        base1 = (cc1 // 5) * 3200 + (cc1 % 5) * 128

        def mstep(r, heads):
            h0, h1 = heads
            g0v = plsc.load_gather(sw, [base0 + jnp.minimum(h0, 127)])
            hs0 = jnp.where(h0 < 128, g0v, 0.0)
            g1v = plsc.load_gather(sw, [base1 + jnp.minimum(h1, 127)])
            hs1 = jnp.where((h1 < 128) & (cc1 < 20), g1v, _NEG)
            gmax = jnp.max(jnp.maximum(hs0, hs1))
            key0 = jnp.where(hs0 == gmax, cc0 * 256 + h0, _BIGI)
            key1 = jnp.where(hs1 == gmax, cc1 * 256 + h1, _BIGI)
            wkey = jnp.min(jnp.minimum(key0, key1))
            wcc = wkey // 256
            wh = wkey % 256
            basew = ((wcc // 5) * 3200 + (wcc % 5) * 128 +
                     jnp.minimum(wh, 127))
            cls_val = jnp.where(gmax > 0.25,
                                (wcc + 1).astype(jnp.float32), 0.0)
            plsc.store_scatter(oM, [splat(r)],
                               jnp.full((16,), cls_val), mask=lane0)
            plsc.store_scatter(oM, [splat(256 + r)],
                               jnp.full((16,), gmax), mask=lane0)
            for a in range(1, 5):
                va = plsc.load_gather(sw, [splat(basew + a * 640)])
                plsc.store_scatter(oM, [splat((a + 1) * 256 + r)], va,
                                   mask=lane0)
            h0n = h0 + (cc0 == wcc).astype(jnp.int32)
            h1n = h1 + (cc1 == wcc).astype(jnp.int32)
            return (h0n, h1n)

        zi = jnp.zeros((16,), jnp.int32)
        lax.fori_loop(0, _TOP_K, mstep, (zi, zi))
        for a in range(6):
            pltpu.sync_copy(oM.at[pl.ds(a * 256, 256)], out_hbm.at[b, a])


def _sc_nms(scores_t, boxes_t, interpret=False):
    return pl.kernel(
        _sc_nms_body,
        out_type=jax.ShapeDtypeStruct((_B, 6, 256), jnp.float32),
        mesh=plsc.VectorSubcoreMesh(core_axis_name="c", subcore_axis_name="s"),
        compiler_params=pltpu.CompilerParams(use_tc_tiling_on_sc=False,
                                             needs_layout_passes=False),
        scratch_types=[
            pltpu.VMEM((5 * _NP,), jnp.float32),
            pltpu.VMEM((4 * _NP,), jnp.float32),
            pltpu.VMEM((_NB,), jnp.float32),
            pltpu.VMEM((640,), jnp.float32),
            pltpu.VMEM((640,), jnp.float32),
            pltpu.VMEM((640,), jnp.float32),
            pltpu.VMEM((640,), jnp.float32),
            pltpu.VMEM((640,), jnp.float32),
            pltpu.VMEM((1536,), jnp.float32),
            pltpu.VMEM_SHARED((16, 3200), jnp.float32),
        ],
        interpret=interpret,
    )(scores_t, boxes_t)


def _merge_body(sS_ref, sY0_ref, sX0_ref, sY1_ref, sX1_ref, out_ref, merged):
    lane128 = lax.broadcasted_iota(jnp.int32, (_B, _CP, 128), 2)
    row_iota = lax.broadcasted_iota(jnp.int32, (_B, _CP, 1), 1)
    sS = sS_ref[...]
    cls_e = jnp.where(sS > 0.25, (row_iota + 1).astype(jnp.float32), 0.0)
    sY0 = sY0_ref[...]
    sX0 = sX0_ref[...]
    sY1 = sY1_ref[...]
    sX1 = sX1_ref[...]

    row8 = lax.broadcasted_iota(jnp.int32, (_B, 8, 256), 1)
    lane256 = lax.broadcasted_iota(jnp.int32, (_B, 8, 256), 2)
    merged[...] = jnp.zeros((_B, 8, 256), jnp.float32)

    def mstep(r, heads):
        hoh = lane128 == heads
        hs = jnp.sum(jnp.where(hoh, sS, 0.0), axis=2, keepdims=True)
        best = jnp.max(hs, axis=1, keepdims=True)
        flat = row_iota * _MAX_NMS + heads
        wflat = jnp.min(jnp.where(hs == best, flat, _BIGI), axis=1,
                        keepdims=True)
        wrow = flat == wflat
        woh = (wrow & hoh).astype(jnp.float32)
        vals = [jnp.sum(jnp.sum(woh * a, axis=2, keepdims=True), axis=1,
                        keepdims=True)
                for a in (cls_e, sS, sY0, sX0, sY1, sX1)]
        col = jnp.zeros((_B, 8, 256), jnp.float32)
        for k, v in enumerate(vals):
            col = col + jnp.where(row8 == k, v, 0.0)
        merged[...] = jnp.where(lane256 == r, col, merged[...])
        return heads + wrow.astype(jnp.int32)

    lax.fori_loop(0, _TOP_K, mstep, jnp.zeros((_B, _CP, 1), jnp.int32))
    out_ref[...] = merged[...]


def _merge(sS, sY0, sX0, sY1, sX1, interpret=False):
    return pl.pallas_call(
        _merge_body,
        out_shape=jax.ShapeDtypeStruct((_B, 8, 256), jnp.float32),
        scratch_shapes=[pltpu.VMEM((_B, 8, 256), jnp.float32)],
        interpret=interpret,
    )(sS, sY0, sX0, sY1, sX1)


def kernel(scores_pred, boxes_pred, _interpret=False):
    scores_t = jnp.transpose(scores_pred[:, :, 1:], (0, 2, 1))
    scores_t = jnp.pad(scores_t, ((0, 0), (0, 0), (0, _NP - _N)))
    scores_t = scores_t.reshape(_B, 4, 5, _NP)
    boxes_t = jnp.transpose(boxes_pred, (0, 2, 1))
    boxes_t = jnp.pad(boxes_t, ((0, 0), (0, 0), (0, _NP - _N)))
    res = _sc_nms(scores_t, boxes_t, interpret=_interpret)
    cls = res[:, 0, :_TOP_K]
    score = res[:, 1, :_TOP_K]
    top_scores = jnp.stack([cls, score], axis=-1)
    top_boxes = jnp.transpose(res[:, 2:6, :_TOP_K], (0, 2, 1))
    return top_scores, top_boxes
